# Initial kernel scaffold; baseline (speedup 1.0000x reference)
import functools

import jax
import jax.numpy as jnp
from jax import lax
from jax.experimental import pallas as pl
from jax.experimental.pallas import tpu as pltpu
from jax.experimental.pallas import tpu_sc as plsc

N = 10000
E = 320000
NW = 32
NSUB = 16
CH = 80
CHUNKS = (E // NW) // CH
NPAD = 10240
SEG = NPAD // NSUB
BLK = 1000

_f32 = jnp.float32


def _mesh():
    return plsc.VectorSubcoreMesh(core_axis_name="c", subcore_axis_name="s")


def _sc_degree(row3):
    @functools.partial(
        pl.kernel,
        out_type=jax.ShapeDtypeStruct((2, NPAD), _f32),
        mesh=_mesh(),
        scratch_types=[
            pltpu.VMEM((CHUNKS, CH), jnp.int32),
            pltpu.VMEM((CH,), _f32),
            pltpu.VMEM((SEG,), _f32),
            pltpu.VMEM_SHARED((NPAD,), _f32),
        ],
    )
    def k(row_hbm, out_hbm, ridx, ones_v, zbuf, acc):
        cid = lax.axis_index("c")
        sid = lax.axis_index("s")
        wid = cid * NSUB + sid
        for t in range(CH // 16):
            ones_v[pl.ds(t * 16, 16)] = jnp.ones((16,), _f32)

        def zb(i, c):
            zbuf[pl.ds(i * 16, 16)] = jnp.zeros((16,), _f32)
            return c

        lax.fori_loop(0, SEG // 16, zb, 0)
        pltpu.sync_copy(row_hbm.at[wid], ridx)
        base = sid * SEG
        pltpu.sync_copy(zbuf, acc.at[pl.ds(base, SEG)])
        plsc.subcore_barrier()

        def body(j, c):
            pltpu.sync_copy(ones_v, acc.at[ridx.at[j]], add=True)
            return c

        lax.fori_loop(0, CHUNKS, body, 0)
        plsc.subcore_barrier()
        pltpu.sync_copy(acc.at[pl.ds(base, SEG)],
                        out_hbm.at[cid].at[pl.ds(base, SEG)])

    return k(row3)


def _sc_prop(v, row3, col3, d):
    @functools.partial(
        pl.kernel,
        out_type=jax.ShapeDtypeStruct((2, NPAD, d), _f32),
        mesh=_mesh(),
        scratch_types=[
            pltpu.VMEM((CHUNKS, CH), jnp.int32),
            pltpu.VMEM((CHUNKS, CH), jnp.int32),
            pltpu.VMEM((CH, d), _f32),
            pltpu.VMEM_SHARED((NPAD, d), _f32),
        ],
    )
    def k(v_hbm, row_hbm, col_hbm, out_hbm, ridx, cidx, gbuf, acc):
        cid = lax.axis_index("c")
        sid = lax.axis_index("s")
        wid = cid * NSUB + sid
        pltpu.sync_copy(row_hbm.at[wid], ridx)
        pltpu.sync_copy(col_hbm.at[wid], cidx)

        def zrow(i, c):
            for t in range(d // 16):
                gbuf[i, pl.ds(t * 16, 16)] = jnp.zeros((16,), _f32)
            return c

        lax.fori_loop(0, CH, zrow, 0)
        base = sid * SEG
        for t in range(SEG // CH):
            pltpu.sync_copy(gbuf, acc.at[pl.ds(base + t * CH, CH)])
        plsc.subcore_barrier()

        def body(j, c):
            pltpu.sync_copy(v_hbm.at[cidx.at[j]], gbuf)
            pltpu.sync_copy(gbuf, acc.at[ridx.at[j]], add=True)
            return c

        lax.fori_loop(0, CHUNKS, body, 0)
        plsc.subcore_barrier()
        pltpu.sync_copy(acc.at[pl.ds(base, SEG)],
                        out_hbm.at[cid].at[pl.ds(base, SEG)])

    return k(v, row3, col3)


def _tc_scale(deg3, x):
    def body(d0, d1, xr, dinv_o, v1_o):
        d = d0[0] + d1[0]
        dinv = lax.rsqrt(d)
        dinv_o[...] = dinv
        v1_o[...] = xr[...] * dinv

    return pl.pallas_call(
        body,
        grid=(N // BLK,),
        in_specs=[
            pl.BlockSpec((1, BLK, 1), lambda j: (0, j, 0)),
            pl.BlockSpec((1, BLK, 1), lambda j: (1, j, 0)),
            pl.BlockSpec((BLK, 128), lambda j: (j, 0)),
        ],
        out_specs=[
            pl.BlockSpec((BLK, 1), lambda j: (j, 0)),
            pl.BlockSpec((BLK, 128), lambda j: (j, 0)),
        ],
        out_shape=[
            jax.ShapeDtypeStruct((N, 1), _f32),
            jax.ShapeDtypeStruct((N, 128), _f32),
        ],
    )(deg3, deg3, x)


def _tc_layer1(u1, dinv, W1, b1, W2):
    def body(p0, p1, s_r, w1_r, b1_r, w2_r, v2_o):
        s = s_r[...]
        p = (p0[0] + p1[0]) * s
        h = jnp.maximum(
            jnp.dot(p, w1_r[...], preferred_element_type=_f32) + b1_r[...], 0.0)
        v2_o[...] = jnp.dot(h, w2_r[...], preferred_element_type=_f32) * s

    return pl.pallas_call(
        body,
        grid=(N // BLK,),
        in_specs=[
            pl.BlockSpec((1, BLK, 128), lambda j: (0, j, 0)),
            pl.BlockSpec((1, BLK, 128), lambda j: (1, j, 0)),
            pl.BlockSpec((BLK, 1), lambda j: (j, 0)),
            pl.BlockSpec((128, 256), lambda j: (0, 0)),
            pl.BlockSpec((1, 256), lambda j: (0, 0)),
            pl.BlockSpec((256, 128), lambda j: (0, 0)),
        ],
        out_specs=pl.BlockSpec((BLK, 128), lambda j: (j, 0)),
        out_shape=jax.ShapeDtypeStruct((N, 128), _f32),
    )(u1, u1, dinv, W1, b1, W2)


def _tc_layer2(u2, dinv, b2, W3):
    def body(p0, p1, s_r, b2_r, w3_r, v3_o):
        s = s_r[...]
        h = jnp.maximum((p0[0] + p1[0]) * s + b2_r[...], 0.0)
        v3_o[...] = jnp.dot(h, w3_r[...], preferred_element_type=_f32) * s

    return pl.pallas_call(
        body,
        grid=(N // BLK,),
        in_specs=[
            pl.BlockSpec((1, BLK, 128), lambda j: (0, j, 0)),
            pl.BlockSpec((1, BLK, 128), lambda j: (1, j, 0)),
            pl.BlockSpec((BLK, 1), lambda j: (j, 0)),
            pl.BlockSpec((1, 128), lambda j: (0, 0)),
            pl.BlockSpec((128, 16), lambda j: (0, 0)),
        ],
        out_specs=pl.BlockSpec((BLK, 16), lambda j: (j, 0)),
        out_shape=jax.ShapeDtypeStruct((N, 16), _f32),
    )(u2, u2, dinv, b2, W3)


def _tc_softmax(u3, dinv, b3):
    def body(p0, p1, s_r, b3_r, o):
        z = (p0[0] + p1[0]) * s_r[...] + b3_r[...]
        z = z - jnp.max(z, axis=1, keepdims=True)
        e = jnp.exp(z)
        o[...] = e / jnp.sum(e, axis=1, keepdims=True)

    return pl.pallas_call(
        body,
        grid=(N // BLK,),
        in_specs=[
            pl.BlockSpec((1, BLK, 16), lambda j: (0, j, 0)),
            pl.BlockSpec((1, BLK, 16), lambda j: (1, j, 0)),
            pl.BlockSpec((BLK, 1), lambda j: (j, 0)),
            pl.BlockSpec((1, 16), lambda j: (0, 0)),
        ],
        out_specs=pl.BlockSpec((BLK, 16), lambda j: (j, 0)),
        out_shape=jax.ShapeDtypeStruct((N, 16), _f32),
    )(u3, u3, dinv, b3)


def kernel(x, edge_index, W1, b1, W2, b2, W3, b3):
    row3 = edge_index[0].reshape(NW, CHUNKS, CH)
    col3 = edge_index[1].reshape(NW, CHUNKS, CH)
    degs = _sc_degree(row3)
    deg3 = degs[:, :, None]
    dinv, v1 = _tc_scale(deg3, x)
    u1 = _sc_prop(v1, row3, col3, 128)
    v2 = _tc_layer1(u1, dinv, W1, b1.reshape(1, -1), W2)
    u2 = _sc_prop(v2, row3, col3, 128)
    v3 = _tc_layer2(u2, dinv, b2.reshape(1, -1), W3)
    u3 = _sc_prop(v3, row3, col3, 16)
    return _tc_softmax(u3, dinv, b3.reshape(1, -1))


# speedup vs baseline: 16.5374x; 16.5374x over previous
import functools

import jax
import jax.numpy as jnp
from jax import lax
from jax.experimental import pallas as pl
from jax.experimental.pallas import tpu as pltpu
from jax.experimental.pallas import tpu_sc as plsc

N = 10000
E = 320000
NW = 32
NSUB = 16
CH = 80
CHUNKS = (E // NW) // CH
NPAD = 10240
SEG = NPAD // NSUB
BLK = 1000

_f32 = jnp.float32


def _mesh():
    return plsc.VectorSubcoreMesh(core_axis_name="c", subcore_axis_name="s")


def _sc_degree(row3):
    @functools.partial(
        pl.kernel,
        out_type=jax.ShapeDtypeStruct((2, NPAD), _f32),
        mesh=_mesh(),
        scratch_types=[
            pltpu.VMEM((CHUNKS, CH), jnp.int32),
            pltpu.VMEM((CH,), _f32),
            pltpu.VMEM((SEG,), _f32),
            pltpu.VMEM_SHARED((NPAD,), _f32),
        ],
    )
    def k(row_hbm, out_hbm, ridx, ones_v, zbuf, acc):
        cid = lax.axis_index("c")
        sid = lax.axis_index("s")
        wid = cid * NSUB + sid
        for t in range(CH // 16):
            ones_v[pl.ds(t * 16, 16)] = jnp.ones((16,), _f32)

        def zb(i, c):
            zbuf[pl.ds(i * 16, 16)] = jnp.zeros((16,), _f32)
            return c

        lax.fori_loop(0, SEG // 16, zb, 0)
        pltpu.sync_copy(row_hbm.at[wid], ridx)
        base = sid * SEG
        pltpu.sync_copy(zbuf, acc.at[pl.ds(base, SEG)])
        plsc.subcore_barrier()

        def body(j, c):
            pltpu.sync_copy(ones_v, acc.at[ridx.at[j]], add=True)
            return c

        lax.fori_loop(0, CHUNKS, body, 0)
        plsc.subcore_barrier()
        pltpu.sync_copy(acc.at[pl.ds(base, SEG)],
                        out_hbm.at[cid].at[pl.ds(base, SEG)])

    return k(row3)


def _sc_prop(v, row3, col3, d):
    @functools.partial(
        pl.kernel,
        out_type=jax.ShapeDtypeStruct((2, NPAD, d), _f32),
        mesh=_mesh(),
        compiler_params=pltpu.CompilerParams(use_tc_tiling_on_sc=False),
        scratch_types=[
            pltpu.VMEM((CHUNKS, CH), jnp.int32),
            pltpu.VMEM((CHUNKS, CH), jnp.int32),
            pltpu.VMEM((CH, d), _f32),
            pltpu.VMEM_SHARED((NPAD, d), _f32),
        ],
    )
    def k(v_hbm, row_hbm, col_hbm, out_hbm, ridx, cidx, gbuf, acc):
        cid = lax.axis_index("c")
        sid = lax.axis_index("s")
        wid = cid * NSUB + sid
        pltpu.sync_copy(row_hbm.at[wid], ridx)
        pltpu.sync_copy(col_hbm.at[wid], cidx)

        def zrow(i, c):
            for t in range(d // 16):
                gbuf[i, pl.ds(t * 16, 16)] = jnp.zeros((16,), _f32)
            return c

        lax.fori_loop(0, CH, zrow, 0)
        base = sid * SEG
        for t in range(SEG // CH):
            pltpu.sync_copy(gbuf, acc.at[pl.ds(base + t * CH, CH)])
        plsc.subcore_barrier()

        def body(j, c):
            pltpu.sync_copy(v_hbm.at[cidx.at[j]], gbuf)
            pltpu.sync_copy(gbuf, acc.at[ridx.at[j]], add=True)
            return c

        lax.fori_loop(0, CHUNKS, body, 0)
        plsc.subcore_barrier()
        pltpu.sync_copy(acc.at[pl.ds(base, SEG)],
                        out_hbm.at[cid].at[pl.ds(base, SEG)])

    return k(v, row3, col3)


def _tc_scale(deg3, x):
    def body(d0, d1, xr, dinv_o, v1_o):
        d = d0[0] + d1[0]
        dinv = lax.rsqrt(d)
        dinv_o[...] = dinv
        v1_o[...] = xr[...] * dinv

    return pl.pallas_call(
        body,
        grid=(N // BLK,),
        in_specs=[
            pl.BlockSpec((1, BLK, 1), lambda j: (0, j, 0)),
            pl.BlockSpec((1, BLK, 1), lambda j: (1, j, 0)),
            pl.BlockSpec((BLK, 128), lambda j: (j, 0)),
        ],
        out_specs=[
            pl.BlockSpec((BLK, 1), lambda j: (j, 0)),
            pl.BlockSpec((BLK, 128), lambda j: (j, 0)),
        ],
        out_shape=[
            jax.ShapeDtypeStruct((N, 1), _f32),
            jax.ShapeDtypeStruct((N, 128), _f32),
        ],
    )(deg3, deg3, x)


def _tc_layer1(u1, dinv, W1, b1, W2):
    def body(p0, p1, s_r, w1_r, b1_r, w2_r, v2_o):
        s = s_r[...]
        p = (p0[0] + p1[0]) * s
        h = jnp.maximum(
            jnp.dot(p, w1_r[...], preferred_element_type=_f32) + b1_r[...], 0.0)
        v2_o[...] = jnp.dot(h, w2_r[...], preferred_element_type=_f32) * s

    return pl.pallas_call(
        body,
        grid=(N // BLK,),
        in_specs=[
            pl.BlockSpec((1, BLK, 128), lambda j: (0, j, 0)),
            pl.BlockSpec((1, BLK, 128), lambda j: (1, j, 0)),
            pl.BlockSpec((BLK, 1), lambda j: (j, 0)),
            pl.BlockSpec((128, 256), lambda j: (0, 0)),
            pl.BlockSpec((1, 256), lambda j: (0, 0)),
            pl.BlockSpec((256, 128), lambda j: (0, 0)),
        ],
        out_specs=pl.BlockSpec((BLK, 128), lambda j: (j, 0)),
        out_shape=jax.ShapeDtypeStruct((N, 128), _f32),
    )(u1, u1, dinv, W1, b1, W2)


def _tc_layer2(u2, dinv, b2, W3):
    def body(p0, p1, s_r, b2_r, w3_r, v3_o):
        s = s_r[...]
        h = jnp.maximum((p0[0] + p1[0]) * s + b2_r[...], 0.0)
        v3_o[...] = jnp.dot(h, w3_r[...], preferred_element_type=_f32) * s

    return pl.pallas_call(
        body,
        grid=(N // BLK,),
        in_specs=[
            pl.BlockSpec((1, BLK, 128), lambda j: (0, j, 0)),
            pl.BlockSpec((1, BLK, 128), lambda j: (1, j, 0)),
            pl.BlockSpec((BLK, 1), lambda j: (j, 0)),
            pl.BlockSpec((1, 128), lambda j: (0, 0)),
            pl.BlockSpec((128, 16), lambda j: (0, 0)),
        ],
        out_specs=pl.BlockSpec((BLK, 16), lambda j: (j, 0)),
        out_shape=jax.ShapeDtypeStruct((N, 16), _f32),
    )(u2, u2, dinv, b2, W3)


def _tc_softmax(u3, dinv, b3):
    def body(p0, p1, s_r, b3_r, o):
        z = (p0[0] + p1[0]) * s_r[...] + b3_r[...]
        z = z - jnp.max(z, axis=1, keepdims=True)
        e = jnp.exp(z)
        o[...] = e / jnp.sum(e, axis=1, keepdims=True)

    return pl.pallas_call(
        body,
        grid=(N // BLK,),
        in_specs=[
            pl.BlockSpec((1, BLK, 16), lambda j: (0, j, 0)),
            pl.BlockSpec((1, BLK, 16), lambda j: (1, j, 0)),
            pl.BlockSpec((BLK, 1), lambda j: (j, 0)),
            pl.BlockSpec((1, 16), lambda j: (0, 0)),
        ],
        out_specs=pl.BlockSpec((BLK, 16), lambda j: (j, 0)),
        out_shape=jax.ShapeDtypeStruct((N, 16), _f32),
    )(u3, u3, dinv, b3)


def kernel(x, edge_index, W1, b1, W2, b2, W3, b3):
    row3 = edge_index[0].reshape(NW, CHUNKS, CH)
    col3 = edge_index[1].reshape(NW, CHUNKS, CH)
    degs = _sc_degree(row3)
    deg3 = degs[:, :, None]
    dinv, v1 = _tc_scale(deg3, x)
    u1 = _sc_prop(v1, row3, col3, 128)
    v2 = _tc_layer1(u1, dinv, W1, b1.reshape(1, -1), W2)
    u2 = _sc_prop(v2, row3, col3, 128)
    v3 = _tc_layer2(u2, dinv, b2.reshape(1, -1), W3)
    u3 = _sc_prop(v3, row3, col3, 16)
    return _tc_softmax(u3, dinv, b3.reshape(1, -1))


# speedup vs baseline: 22.5571x; 1.3640x over previous
import functools

import jax
import jax.numpy as jnp
from jax import lax
from jax.experimental import pallas as pl
from jax.experimental.pallas import tpu as pltpu
from jax.experimental.pallas import tpu_sc as plsc

N = 10000
E = 320000
NW = 32
NSUB = 16
CH = 80
CHUNKS = (E // NW) // CH
NPAD = 10240
SEG = NPAD // NSUB
BLK = 1000

_f32 = jnp.float32


def _mesh():
    return plsc.VectorSubcoreMesh(core_axis_name="c", subcore_axis_name="s")


def _sc_degree(row3):
    @functools.partial(
        pl.kernel,
        out_type=jax.ShapeDtypeStruct((2, NPAD), _f32),
        mesh=_mesh(),
        scratch_types=[
            pltpu.VMEM((CHUNKS, CH), jnp.int32),
            pltpu.VMEM((CH,), _f32),
            pltpu.VMEM((SEG,), _f32),
            pltpu.VMEM_SHARED((NPAD,), _f32),
        ],
    )
    def k(row_hbm, out_hbm, ridx, ones_v, zbuf, acc):
        cid = lax.axis_index("c")
        sid = lax.axis_index("s")
        wid = cid * NSUB + sid
        for t in range(CH // 16):
            ones_v[pl.ds(t * 16, 16)] = jnp.ones((16,), _f32)

        def zb(i, c):
            zbuf[pl.ds(i * 16, 16)] = jnp.zeros((16,), _f32)
            return c

        lax.fori_loop(0, SEG // 16, zb, 0)
        pltpu.sync_copy(row_hbm.at[wid], ridx)
        base = sid * SEG
        pltpu.sync_copy(zbuf, acc.at[pl.ds(base, SEG)])
        plsc.subcore_barrier()

        def body(j, c):
            pltpu.sync_copy(ones_v, acc.at[ridx.at[j]], add=True)
            return c

        lax.fori_loop(0, CHUNKS, body, 0)
        plsc.subcore_barrier()
        pltpu.sync_copy(acc.at[pl.ds(base, SEG)],
                        out_hbm.at[cid].at[pl.ds(base, SEG)])

    return k(row3)


def _sc_prop(v, row3, col3, d):
    NBUF = 2 if d > 16 else 5
    MAIN = (CHUNKS // NBUF) * NBUF

    @functools.partial(
        pl.kernel,
        out_type=jax.ShapeDtypeStruct((2, NPAD, d), _f32),
        mesh=_mesh(),
        compiler_params=pltpu.CompilerParams(use_tc_tiling_on_sc=False),
        scratch_types=[
            pltpu.VMEM((CHUNKS, CH), jnp.int32),
            pltpu.VMEM((CHUNKS, CH), jnp.int32),
            [pltpu.VMEM((CH, d), _f32) for _ in range(NBUF)],
            pltpu.VMEM_SHARED((NPAD, d), _f32),
            pltpu.SemaphoreType.DMA((NBUF,)),
            pltpu.SemaphoreType.DMA((NBUF,)),
        ],
    )
    def k(v_hbm, row_hbm, col_hbm, out_hbm, ridx, cidx, gbufs, acc,
          gsem, ssem):
        cid = lax.axis_index("c")
        sid = lax.axis_index("s")
        wid = cid * NSUB + sid
        pltpu.sync_copy(row_hbm.at[wid], ridx)
        pltpu.sync_copy(col_hbm.at[wid], cidx)

        def zrow(i, c):
            for t in range(d // 16):
                gbufs[0][i, pl.ds(t * 16, 16)] = jnp.zeros((16,), _f32)
            return c

        lax.fori_loop(0, CH, zrow, 0)
        base = sid * SEG
        for t in range(SEG // CH):
            pltpu.sync_copy(gbufs[0], acc.at[pl.ds(base + t * CH, CH)])
        plsc.subcore_barrier()

        def body(i, c):
            j0 = i * NBUF
            gd = [pltpu.async_copy(v_hbm.at[cidx.at[j0 + b]],
                                   gbufs[b], gsem.at[b])
                  for b in range(NBUF)]
            sd = []
            for b in range(NBUF):
                gd[b].wait()
                sd.append(pltpu.async_copy(gbufs[b],
                                           acc.at[ridx.at[j0 + b]],
                                           ssem.at[b], add=True))
            for b in range(NBUF):
                sd[b].wait()
            return c

        lax.fori_loop(0, MAIN // NBUF, body, 0)
        for j in range(MAIN, CHUNKS):
            pltpu.sync_copy(v_hbm.at[cidx.at[j]], gbufs[0])
            pltpu.sync_copy(gbufs[0], acc.at[ridx.at[j]], add=True)
        plsc.subcore_barrier()
        pltpu.sync_copy(acc.at[pl.ds(base, SEG)],
                        out_hbm.at[cid].at[pl.ds(base, SEG)])

    return k(v, row3, col3)


def _tc_scale(deg3, x):
    def body(d0, d1, xr, dinv_o, v1_o):
        d = d0[0] + d1[0]
        dinv = lax.rsqrt(d)
        dinv_o[...] = dinv
        v1_o[...] = xr[...] * dinv

    return pl.pallas_call(
        body,
        grid=(N // BLK,),
        in_specs=[
            pl.BlockSpec((1, BLK, 1), lambda j: (0, j, 0)),
            pl.BlockSpec((1, BLK, 1), lambda j: (1, j, 0)),
            pl.BlockSpec((BLK, 128), lambda j: (j, 0)),
        ],
        out_specs=[
            pl.BlockSpec((BLK, 1), lambda j: (j, 0)),
            pl.BlockSpec((BLK, 128), lambda j: (j, 0)),
        ],
        out_shape=[
            jax.ShapeDtypeStruct((N, 1), _f32),
            jax.ShapeDtypeStruct((N, 128), _f32),
        ],
    )(deg3, deg3, x)


def _tc_layer1(u1, dinv, W1, b1, W2):
    def body(p0, p1, s_r, w1_r, b1_r, w2_r, v2_o):
        s = s_r[...]
        p = (p0[0] + p1[0]) * s
        h = jnp.maximum(
            jnp.dot(p, w1_r[...], preferred_element_type=_f32) + b1_r[...], 0.0)
        v2_o[...] = jnp.dot(h, w2_r[...], preferred_element_type=_f32) * s

    return pl.pallas_call(
        body,
        grid=(N // BLK,),
        in_specs=[
            pl.BlockSpec((1, BLK, 128), lambda j: (0, j, 0)),
            pl.BlockSpec((1, BLK, 128), lambda j: (1, j, 0)),
            pl.BlockSpec((BLK, 1), lambda j: (j, 0)),
            pl.BlockSpec((128, 256), lambda j: (0, 0)),
            pl.BlockSpec((1, 256), lambda j: (0, 0)),
            pl.BlockSpec((256, 128), lambda j: (0, 0)),
        ],
        out_specs=pl.BlockSpec((BLK, 128), lambda j: (j, 0)),
        out_shape=jax.ShapeDtypeStruct((N, 128), _f32),
    )(u1, u1, dinv, W1, b1, W2)


def _tc_layer2(u2, dinv, b2, W3):
    def body(p0, p1, s_r, b2_r, w3_r, v3_o):
        s = s_r[...]
        h = jnp.maximum((p0[0] + p1[0]) * s + b2_r[...], 0.0)
        v3_o[...] = jnp.dot(h, w3_r[...], preferred_element_type=_f32) * s

    return pl.pallas_call(
        body,
        grid=(N // BLK,),
        in_specs=[
            pl.BlockSpec((1, BLK, 128), lambda j: (0, j, 0)),
            pl.BlockSpec((1, BLK, 128), lambda j: (1, j, 0)),
            pl.BlockSpec((BLK, 1), lambda j: (j, 0)),
            pl.BlockSpec((1, 128), lambda j: (0, 0)),
            pl.BlockSpec((128, 16), lambda j: (0, 0)),
        ],
        out_specs=pl.BlockSpec((BLK, 16), lambda j: (j, 0)),
        out_shape=jax.ShapeDtypeStruct((N, 16), _f32),
    )(u2, u2, dinv, b2, W3)


def _tc_softmax(u3, dinv, b3):
    def body(p0, p1, s_r, b3_r, o):
        z = (p0[0] + p1[0]) * s_r[...] + b3_r[...]
        z = z - jnp.max(z, axis=1, keepdims=True)
        e = jnp.exp(z)
        o[...] = e / jnp.sum(e, axis=1, keepdims=True)

    return pl.pallas_call(
        body,
        grid=(N // BLK,),
        in_specs=[
            pl.BlockSpec((1, BLK, 16), lambda j: (0, j, 0)),
            pl.BlockSpec((1, BLK, 16), lambda j: (1, j, 0)),
            pl.BlockSpec((BLK, 1), lambda j: (j, 0)),
            pl.BlockSpec((1, 16), lambda j: (0, 0)),
        ],
        out_specs=pl.BlockSpec((BLK, 16), lambda j: (j, 0)),
        out_shape=jax.ShapeDtypeStruct((N, 16), _f32),
    )(u3, u3, dinv, b3)


def kernel(x, edge_index, W1, b1, W2, b2, W3, b3):
    row3 = edge_index[0].reshape(NW, CHUNKS, CH)
    col3 = edge_index[1].reshape(NW, CHUNKS, CH)
    degs = _sc_degree(row3)
    deg3 = degs[:, :, None]
    dinv, v1 = _tc_scale(deg3, x)
    u1 = _sc_prop(v1, row3, col3, 128)
    v2 = _tc_layer1(u1, dinv, W1, b1.reshape(1, -1), W2)
    u2 = _sc_prop(v2, row3, col3, 128)
    v3 = _tc_layer2(u2, dinv, b2.reshape(1, -1), W3)
    u3 = _sc_prop(v3, row3, col3, 16)
    return _tc_softmax(u3, dinv, b3.reshape(1, -1))


# speedup vs baseline: 23.0777x; 1.0231x over previous
import functools

import jax
import jax.numpy as jnp
from jax import lax
from jax.experimental import pallas as pl
from jax.experimental.pallas import tpu as pltpu
from jax.experimental.pallas import tpu_sc as plsc

N = 10000
E = 320000
NW = 32
NSUB = 16
CH = 80
CHUNKS = (E // NW) // CH
NPAD = 10240
SEG = NPAD // NSUB
BLK = 1000

_f32 = jnp.float32


def _mesh():
    return plsc.VectorSubcoreMesh(core_axis_name="c", subcore_axis_name="s")


def _sc_degree(row3):
    @functools.partial(
        pl.kernel,
        out_type=jax.ShapeDtypeStruct((2, NPAD), _f32),
        mesh=_mesh(),
        scratch_types=[
            pltpu.VMEM((CHUNKS, CH), jnp.int32),
            pltpu.VMEM((CH,), _f32),
            pltpu.VMEM((SEG,), _f32),
            pltpu.VMEM_SHARED((NPAD,), _f32),
        ],
    )
    def k(row_hbm, out_hbm, ridx, ones_v, zbuf, acc):
        cid = lax.axis_index("c")
        sid = lax.axis_index("s")
        wid = cid * NSUB + sid
        for t in range(CH // 16):
            ones_v[pl.ds(t * 16, 16)] = jnp.ones((16,), _f32)

        def zb(i, c):
            zbuf[pl.ds(i * 16, 16)] = jnp.zeros((16,), _f32)
            return c

        lax.fori_loop(0, SEG // 16, zb, 0)
        pltpu.sync_copy(row_hbm.at[wid], ridx)
        base = sid * SEG
        pltpu.sync_copy(zbuf, acc.at[pl.ds(base, SEG)])
        plsc.subcore_barrier()

        def body(j, c):
            pltpu.sync_copy(ones_v, acc.at[ridx.at[j]], add=True)
            return c

        lax.fori_loop(0, CHUNKS, body, 0)
        plsc.subcore_barrier()
        pltpu.sync_copy(acc.at[pl.ds(base, SEG)],
                        out_hbm.at[cid].at[pl.ds(base, SEG)])

    return k(row3)


def _sc_prop(v, row3, col3, d):
    NBUF = 2 if d > 16 else 5
    MAIN = (CHUNKS // NBUF) * NBUF

    @functools.partial(
        pl.kernel,
        out_type=jax.ShapeDtypeStruct((2, NPAD, d), _f32),
        mesh=_mesh(),
        compiler_params=pltpu.CompilerParams(use_tc_tiling_on_sc=False),
        scratch_types=[
            pltpu.VMEM((CHUNKS, CH), jnp.int32),
            pltpu.VMEM((CHUNKS, CH), jnp.int32),
            [pltpu.VMEM((CH, d), _f32) for _ in range(NBUF)],
            pltpu.VMEM_SHARED((NPAD, d), _f32),
            pltpu.SemaphoreType.DMA((NBUF,)),
            pltpu.SemaphoreType.DMA((NBUF,)),
        ],
    )
    def k(v_hbm, row_hbm, col_hbm, out_hbm, ridx, cidx, gbufs, acc,
          gsem, ssem):
        cid = lax.axis_index("c")
        sid = lax.axis_index("s")
        wid = cid * NSUB + sid
        pltpu.sync_copy(row_hbm.at[wid], ridx)
        pltpu.sync_copy(col_hbm.at[wid], cidx)

        def zrow(i, c):
            for t in range(d // 16):
                gbufs[0][i, pl.ds(t * 16, 16)] = jnp.zeros((16,), _f32)
            return c

        lax.fori_loop(0, CH, zrow, 0)
        base = sid * SEG
        for t in range(SEG // CH):
            pltpu.sync_copy(gbufs[0], acc.at[pl.ds(base + t * CH, CH)])
        plsc.subcore_barrier()

        for b in range(NBUF):
            pltpu.async_copy(v_hbm.at[cidx.at[b]], gbufs[b], gsem.at[b])

        def body(i, c):
            j0 = i * NBUF
            for b in range(NBUF):
                pltpu.make_async_copy(v_hbm.at[cidx.at[j0 + b]],
                                      gbufs[b], gsem.at[b]).wait()
                pltpu.async_copy(gbufs[b], acc.at[ridx.at[j0 + b]],
                                 ssem.at[b], add=True)

            @pl.when(j0 + NBUF < MAIN)
            def _():
                for b in range(NBUF):
                    pltpu.make_async_copy(gbufs[b], acc.at[ridx.at[j0 + b]],
                                          ssem.at[b]).wait()
                    pltpu.async_copy(v_hbm.at[cidx.at[j0 + NBUF + b]],
                                     gbufs[b], gsem.at[b])
            return c

        lax.fori_loop(0, MAIN // NBUF, body, 0)
        for b in range(NBUF):
            pltpu.make_async_copy(gbufs[b], acc.at[ridx.at[b]],
                                  ssem.at[b]).wait()
        for j in range(MAIN, CHUNKS):
            pltpu.sync_copy(v_hbm.at[cidx.at[j]], gbufs[0])
            pltpu.sync_copy(gbufs[0], acc.at[ridx.at[j]], add=True)
        plsc.subcore_barrier()
        pltpu.sync_copy(acc.at[pl.ds(base, SEG)],
                        out_hbm.at[cid].at[pl.ds(base, SEG)])

    return k(v, row3, col3)


def _tc_scale(deg3, x):
    def body(d0, d1, xr, dinv_o, v1_o):
        d = d0[0] + d1[0]
        dinv = lax.rsqrt(d)
        dinv_o[...] = dinv
        v1_o[...] = xr[...] * dinv

    return pl.pallas_call(
        body,
        grid=(N // BLK,),
        in_specs=[
            pl.BlockSpec((1, BLK, 1), lambda j: (0, j, 0)),
            pl.BlockSpec((1, BLK, 1), lambda j: (1, j, 0)),
            pl.BlockSpec((BLK, 128), lambda j: (j, 0)),
        ],
        out_specs=[
            pl.BlockSpec((BLK, 1), lambda j: (j, 0)),
            pl.BlockSpec((BLK, 128), lambda j: (j, 0)),
        ],
        out_shape=[
            jax.ShapeDtypeStruct((N, 1), _f32),
            jax.ShapeDtypeStruct((N, 128), _f32),
        ],
    )(deg3, deg3, x)


def _tc_layer1(u1, dinv, W1, b1, W2):
    def body(p0, p1, s_r, w1_r, b1_r, w2_r, v2_o):
        s = s_r[...]
        p = (p0[0] + p1[0]) * s
        h = jnp.maximum(
            jnp.dot(p, w1_r[...], preferred_element_type=_f32) + b1_r[...], 0.0)
        v2_o[...] = jnp.dot(h, w2_r[...], preferred_element_type=_f32) * s

    return pl.pallas_call(
        body,
        grid=(N // BLK,),
        in_specs=[
            pl.BlockSpec((1, BLK, 128), lambda j: (0, j, 0)),
            pl.BlockSpec((1, BLK, 128), lambda j: (1, j, 0)),
            pl.BlockSpec((BLK, 1), lambda j: (j, 0)),
            pl.BlockSpec((128, 256), lambda j: (0, 0)),
            pl.BlockSpec((1, 256), lambda j: (0, 0)),
            pl.BlockSpec((256, 128), lambda j: (0, 0)),
        ],
        out_specs=pl.BlockSpec((BLK, 128), lambda j: (j, 0)),
        out_shape=jax.ShapeDtypeStruct((N, 128), _f32),
    )(u1, u1, dinv, W1, b1, W2)


def _tc_layer2(u2, dinv, b2, W3):
    def body(p0, p1, s_r, b2_r, w3_r, v3_o):
        s = s_r[...]
        h = jnp.maximum((p0[0] + p1[0]) * s + b2_r[...], 0.0)
        v3_o[...] = jnp.dot(h, w3_r[...], preferred_element_type=_f32) * s

    return pl.pallas_call(
        body,
        grid=(N // BLK,),
        in_specs=[
            pl.BlockSpec((1, BLK, 128), lambda j: (0, j, 0)),
            pl.BlockSpec((1, BLK, 128), lambda j: (1, j, 0)),
            pl.BlockSpec((BLK, 1), lambda j: (j, 0)),
            pl.BlockSpec((1, 128), lambda j: (0, 0)),
            pl.BlockSpec((128, 16), lambda j: (0, 0)),
        ],
        out_specs=pl.BlockSpec((BLK, 16), lambda j: (j, 0)),
        out_shape=jax.ShapeDtypeStruct((N, 16), _f32),
    )(u2, u2, dinv, b2, W3)


def _tc_softmax(u3, dinv, b3):
    def body(p0, p1, s_r, b3_r, o):
        z = (p0[0] + p1[0]) * s_r[...] + b3_r[...]
        z = z - jnp.max(z, axis=1, keepdims=True)
        e = jnp.exp(z)
        o[...] = e / jnp.sum(e, axis=1, keepdims=True)

    return pl.pallas_call(
        body,
        grid=(N // BLK,),
        in_specs=[
            pl.BlockSpec((1, BLK, 16), lambda j: (0, j, 0)),
            pl.BlockSpec((1, BLK, 16), lambda j: (1, j, 0)),
            pl.BlockSpec((BLK, 1), lambda j: (j, 0)),
            pl.BlockSpec((1, 16), lambda j: (0, 0)),
        ],
        out_specs=pl.BlockSpec((BLK, 16), lambda j: (j, 0)),
        out_shape=jax.ShapeDtypeStruct((N, 16), _f32),
    )(u3, u3, dinv, b3)


def kernel(x, edge_index, W1, b1, W2, b2, W3, b3):
    row3 = edge_index[0].reshape(NW, CHUNKS, CH)
    col3 = edge_index[1].reshape(NW, CHUNKS, CH)
    degs = _sc_degree(row3)
    deg3 = degs[:, :, None]
    dinv, v1 = _tc_scale(deg3, x)
    u1 = _sc_prop(v1, row3, col3, 128)
    v2 = _tc_layer1(u1, dinv, W1, b1.reshape(1, -1), W2)
    u2 = _sc_prop(v2, row3, col3, 128)
    v3 = _tc_layer2(u2, dinv, b2.reshape(1, -1), W3)
    u3 = _sc_prop(v3, row3, col3, 16)
    return _tc_softmax(u3, dinv, b3.reshape(1, -1))


# speedup vs baseline: 23.4909x; 1.0179x over previous
import functools

import jax
import jax.numpy as jnp
from jax import lax
from jax.experimental import pallas as pl
from jax.experimental.pallas import tpu as pltpu
from jax.experimental.pallas import tpu_sc as plsc

N = 10000
E = 320000
NW = 32
NSUB = 16
CH = 80
CHUNKS = (E // NW) // CH
NPAD = 10240
SEG = NPAD // NSUB
BLK = 1000

_f32 = jnp.float32


def _mesh():
    return plsc.VectorSubcoreMesh(core_axis_name="c", subcore_axis_name="s")


def _sc_degree(edge3):
    @functools.partial(
        pl.kernel,
        out_type=jax.ShapeDtypeStruct((2, NPAD), _f32),
        mesh=_mesh(),
        scratch_types=[
            pltpu.VMEM((CHUNKS, CH), jnp.int32),
            pltpu.VMEM((CH,), _f32),
            pltpu.VMEM((SEG,), _f32),
            pltpu.VMEM_SHARED((NPAD,), _f32),
        ],
    )
    def k(edge_hbm, out_hbm, ridx, ones_v, zbuf, acc):
        cid = lax.axis_index("c")
        sid = lax.axis_index("s")
        wid = cid * NSUB + sid
        for t in range(CH // 16):
            ones_v[pl.ds(t * 16, 16)] = jnp.ones((16,), _f32)

        def zb(i, c):
            zbuf[pl.ds(i * 16, 16)] = jnp.zeros((16,), _f32)
            return c

        lax.fori_loop(0, SEG // 16, zb, 0)
        pltpu.sync_copy(edge_hbm.at[0].at[wid], ridx)
        base = sid * SEG
        pltpu.sync_copy(zbuf, acc.at[pl.ds(base, SEG)])
        plsc.subcore_barrier()

        def body(j, c):
            pltpu.sync_copy(ones_v, acc.at[ridx.at[j]], add=True)
            return c

        lax.fori_loop(0, CHUNKS, body, 0)
        plsc.subcore_barrier()
        pltpu.sync_copy(acc.at[pl.ds(base, SEG)],
                        out_hbm.at[cid].at[pl.ds(base, SEG)])

    return k(edge3)


def _sc_prop(v, edge3, d):
    NBUF = 2 if d > 16 else 5
    MAIN = (CHUNKS // NBUF) * NBUF

    @functools.partial(
        pl.kernel,
        out_type=jax.ShapeDtypeStruct((2, NPAD, d), _f32),
        mesh=_mesh(),
        compiler_params=pltpu.CompilerParams(use_tc_tiling_on_sc=False),
        scratch_types=[
            pltpu.VMEM((CHUNKS, CH), jnp.int32),
            pltpu.VMEM((CHUNKS, CH), jnp.int32),
            [pltpu.VMEM((CH, d), _f32) for _ in range(NBUF)],
            pltpu.VMEM_SHARED((NPAD, d), _f32),
            pltpu.SemaphoreType.DMA((NBUF,)),
            pltpu.SemaphoreType.DMA((NBUF,)),
        ],
    )
    def k(v_hbm, edge_hbm, out_hbm, ridx, cidx, gbufs, acc,
          gsem, ssem):
        cid = lax.axis_index("c")
        sid = lax.axis_index("s")
        wid = cid * NSUB + sid
        pltpu.sync_copy(edge_hbm.at[0].at[wid], ridx)
        pltpu.sync_copy(edge_hbm.at[1].at[wid], cidx)

        def zrow(i, c):
            for t in range(d // 16):
                gbufs[0][i, pl.ds(t * 16, 16)] = jnp.zeros((16,), _f32)
            return c

        lax.fori_loop(0, CH, zrow, 0)
        base = sid * SEG
        for t in range(SEG // CH):
            pltpu.sync_copy(gbufs[0], acc.at[pl.ds(base + t * CH, CH)])
        plsc.subcore_barrier()

        for b in range(NBUF):
            pltpu.async_copy(v_hbm.at[cidx.at[b]], gbufs[b], gsem.at[b])

        def body(i, c):
            j0 = i * NBUF
            for b in range(NBUF):
                pltpu.make_async_copy(v_hbm.at[cidx.at[j0 + b]],
                                      gbufs[b], gsem.at[b]).wait()
                pltpu.async_copy(gbufs[b], acc.at[ridx.at[j0 + b]],
                                 ssem.at[b], add=True)

            @pl.when(j0 + NBUF < MAIN)
            def _():
                for b in range(NBUF):
                    pltpu.make_async_copy(gbufs[b], acc.at[ridx.at[j0 + b]],
                                          ssem.at[b]).wait()
                    pltpu.async_copy(v_hbm.at[cidx.at[j0 + NBUF + b]],
                                     gbufs[b], gsem.at[b])
            return c

        lax.fori_loop(0, MAIN // NBUF, body, 0)
        for b in range(NBUF):
            pltpu.make_async_copy(gbufs[b], acc.at[ridx.at[b]],
                                  ssem.at[b]).wait()
        for j in range(MAIN, CHUNKS):
            pltpu.sync_copy(v_hbm.at[cidx.at[j]], gbufs[0])
            pltpu.sync_copy(gbufs[0], acc.at[ridx.at[j]], add=True)
        plsc.subcore_barrier()
        pltpu.sync_copy(acc.at[pl.ds(base, SEG)],
                        out_hbm.at[cid].at[pl.ds(base, SEG)])

    return k(v, edge3)


def _tc_scale(deg3, x):
    def body(d0, d1, xr, dinv_o, v1_o):
        d = d0[0] + d1[0]
        dinv = lax.rsqrt(d)
        dinv_o[...] = dinv
        v1_o[...] = xr[...] * dinv

    return pl.pallas_call(
        body,
        grid=(N // BLK,),
        in_specs=[
            pl.BlockSpec((1, BLK, 1), lambda j: (0, j, 0)),
            pl.BlockSpec((1, BLK, 1), lambda j: (1, j, 0)),
            pl.BlockSpec((BLK, 128), lambda j: (j, 0)),
        ],
        out_specs=[
            pl.BlockSpec((BLK, 1), lambda j: (j, 0)),
            pl.BlockSpec((BLK, 128), lambda j: (j, 0)),
        ],
        out_shape=[
            jax.ShapeDtypeStruct((N, 1), _f32),
            jax.ShapeDtypeStruct((N, 128), _f32),
        ],
    )(deg3, deg3, x)


def _tc_layer1(u1, dinv, W1, b1, W2):
    def body(p0, p1, s_r, w1_r, b1_r, w2_r, v2_o):
        s = s_r[...]
        p = (p0[0] + p1[0]) * s
        h = jnp.maximum(
            jnp.dot(p, w1_r[...], preferred_element_type=_f32) + b1_r[...], 0.0)
        v2_o[...] = jnp.dot(h, w2_r[...], preferred_element_type=_f32) * s

    return pl.pallas_call(
        body,
        grid=(N // BLK,),
        in_specs=[
            pl.BlockSpec((1, BLK, 128), lambda j: (0, j, 0)),
            pl.BlockSpec((1, BLK, 128), lambda j: (1, j, 0)),
            pl.BlockSpec((BLK, 1), lambda j: (j, 0)),
            pl.BlockSpec((128, 256), lambda j: (0, 0)),
            pl.BlockSpec((1, 256), lambda j: (0, 0)),
            pl.BlockSpec((256, 128), lambda j: (0, 0)),
        ],
        out_specs=pl.BlockSpec((BLK, 128), lambda j: (j, 0)),
        out_shape=jax.ShapeDtypeStruct((N, 128), _f32),
    )(u1, u1, dinv, W1, b1, W2)


def _tc_layer2(u2, dinv, b2, W3):
    def body(p0, p1, s_r, b2_r, w3_r, v3_o):
        s = s_r[...]
        h = jnp.maximum((p0[0] + p1[0]) * s + b2_r[...], 0.0)
        v3_o[...] = jnp.dot(h, w3_r[...], preferred_element_type=_f32) * s

    return pl.pallas_call(
        body,
        grid=(N // BLK,),
        in_specs=[
            pl.BlockSpec((1, BLK, 128), lambda j: (0, j, 0)),
            pl.BlockSpec((1, BLK, 128), lambda j: (1, j, 0)),
            pl.BlockSpec((BLK, 1), lambda j: (j, 0)),
            pl.BlockSpec((1, 128), lambda j: (0, 0)),
            pl.BlockSpec((128, 16), lambda j: (0, 0)),
        ],
        out_specs=pl.BlockSpec((BLK, 16), lambda j: (j, 0)),
        out_shape=jax.ShapeDtypeStruct((N, 16), _f32),
    )(u2, u2, dinv, b2, W3)


def _tc_softmax(u3, dinv, b3):
    def body(p0, p1, s_r, b3_r, o):
        z = (p0[0] + p1[0]) * s_r[...] + b3_r[...]
        z = z - jnp.max(z, axis=1, keepdims=True)
        e = jnp.exp(z)
        o[...] = e / jnp.sum(e, axis=1, keepdims=True)

    return pl.pallas_call(
        body,
        grid=(N // BLK,),
        in_specs=[
            pl.BlockSpec((1, BLK, 16), lambda j: (0, j, 0)),
            pl.BlockSpec((1, BLK, 16), lambda j: (1, j, 0)),
            pl.BlockSpec((BLK, 1), lambda j: (j, 0)),
            pl.BlockSpec((1, 16), lambda j: (0, 0)),
        ],
        out_specs=pl.BlockSpec((BLK, 16), lambda j: (j, 0)),
        out_shape=jax.ShapeDtypeStruct((N, 16), _f32),
    )(u3, u3, dinv, b3)


def kernel(x, edge_index, W1, b1, W2, b2, W3, b3):
    edge3 = edge_index.reshape(2, NW, CHUNKS, CH)
    degs = _sc_degree(edge3)
    deg3 = degs[:, :, None]
    dinv, v1 = _tc_scale(deg3, x)
    u1 = _sc_prop(v1, edge3, 128)
    v2 = _tc_layer1(u1, dinv, W1, b1.reshape(1, -1), W2)
    u2 = _sc_prop(v2, edge3, 128)
    v3 = _tc_layer2(u2, dinv, b2.reshape(1, -1), W3)
    u3 = _sc_prop(v3, edge3, 16)
    return _tc_softmax(u3, dinv, b3.reshape(1, -1))


# speedup vs baseline: 26.9266x; 1.1463x over previous
import functools

import jax
import jax.numpy as jnp
from jax import lax
from jax.experimental import pallas as pl
from jax.experimental.pallas import tpu as pltpu
from jax.experimental.pallas import tpu_sc as plsc

N = 10000
E = 320000
NW = 32
NSUB = 16
CH = 80
CHUNKS = (E // NW) // CH
NPAD = 10240
SEG = NPAD // NSUB
BLK = 2000

_f32 = jnp.float32


def _mesh():
    return plsc.VectorSubcoreMesh(core_axis_name="c", subcore_axis_name="s")


def _sc_degree(edge3):
    @functools.partial(
        pl.kernel,
        out_type=jax.ShapeDtypeStruct((2, NPAD), _f32),
        mesh=_mesh(),
        scratch_types=[
            pltpu.VMEM((CHUNKS, CH), jnp.int32),
            pltpu.VMEM((CH,), _f32),
            pltpu.VMEM((SEG,), _f32),
            pltpu.VMEM_SHARED((NPAD,), _f32),
        ],
    )
    def k(edge_hbm, out_hbm, ridx, ones_v, zbuf, acc):
        cid = lax.axis_index("c")
        sid = lax.axis_index("s")
        wid = cid * NSUB + sid
        for t in range(CH // 16):
            ones_v[pl.ds(t * 16, 16)] = jnp.ones((16,), _f32)

        def zb(i, c):
            zbuf[pl.ds(i * 16, 16)] = jnp.zeros((16,), _f32)
            return c

        lax.fori_loop(0, SEG // 16, zb, 0)
        pltpu.sync_copy(edge_hbm.at[0].at[wid], ridx)
        base = sid * SEG
        pltpu.sync_copy(zbuf, acc.at[pl.ds(base, SEG)])
        plsc.subcore_barrier()

        def body(j, c):
            pltpu.sync_copy(ones_v, acc.at[ridx.at[j]], add=True)
            return c

        lax.fori_loop(0, CHUNKS, body, 0)
        plsc.subcore_barrier()
        pltpu.sync_copy(acc.at[pl.ds(base, SEG)],
                        out_hbm.at[cid].at[pl.ds(base, SEG)])

    return k(edge3)


def _sc_prop_half(v, edge3):
    HD = 64
    C2 = 2 * CHUNKS
    NBUF = 5

    @functools.partial(
        pl.kernel,
        out_type=jax.ShapeDtypeStruct((2, NPAD, HD), _f32),
        mesh=_mesh(),
        compiler_params=pltpu.CompilerParams(use_tc_tiling_on_sc=False),
        scratch_types=[
            pltpu.VMEM((C2, CH), jnp.int32),
            pltpu.VMEM((C2, CH), jnp.int32),
            [pltpu.VMEM((CH, HD), _f32) for _ in range(NBUF)],
            pltpu.VMEM_SHARED((NPAD, HD), _f32),
            pltpu.SemaphoreType.DMA((NBUF,)),
            pltpu.SemaphoreType.DMA((NBUF,)),
        ],
    )
    def k(v_hbm, edge_hbm, out_hbm, ridx, cidx, gbufs, acc, gsem, ssem):
        cid = lax.axis_index("c")
        sid = lax.axis_index("s")
        vh = v_hbm.at[cid]
        pltpu.sync_copy(edge_hbm.at[0].at[2 * sid], ridx.at[pl.ds(0, CHUNKS)])
        pltpu.sync_copy(edge_hbm.at[0].at[2 * sid + 1],
                        ridx.at[pl.ds(CHUNKS, CHUNKS)])
        pltpu.sync_copy(edge_hbm.at[1].at[2 * sid], cidx.at[pl.ds(0, CHUNKS)])
        pltpu.sync_copy(edge_hbm.at[1].at[2 * sid + 1],
                        cidx.at[pl.ds(CHUNKS, CHUNKS)])

        def zrow(i, c):
            for t in range(HD // 16):
                gbufs[0][i, pl.ds(t * 16, 16)] = jnp.zeros((16,), _f32)
            return c

        lax.fori_loop(0, CH, zrow, 0)
        base = sid * SEG
        for t in range(SEG // CH):
            pltpu.sync_copy(gbufs[0], acc.at[pl.ds(base + t * CH, CH)])
        plsc.subcore_barrier()

        for b in range(NBUF):
            pltpu.async_copy(vh.at[cidx.at[b]], gbufs[b], gsem.at[b])

        def body(i, c):
            j0 = i * NBUF
            for b in range(NBUF):
                pltpu.make_async_copy(vh.at[cidx.at[j0 + b]],
                                      gbufs[b], gsem.at[b]).wait()
                pltpu.async_copy(gbufs[b], acc.at[ridx.at[j0 + b]],
                                 ssem.at[b], add=True)

            @pl.when(j0 + NBUF < C2)
            def _():
                for b in range(NBUF):
                    pltpu.make_async_copy(gbufs[b], acc.at[ridx.at[j0 + b]],
                                          ssem.at[b]).wait()
                    pltpu.async_copy(vh.at[cidx.at[j0 + NBUF + b]],
                                     gbufs[b], gsem.at[b])
            return c

        lax.fori_loop(0, C2 // NBUF, body, 0)
        for b in range(NBUF):
            pltpu.make_async_copy(gbufs[b], acc.at[ridx.at[b]],
                                  ssem.at[b]).wait()
        plsc.subcore_barrier()
        pltpu.sync_copy(acc.at[pl.ds(base, SEG)],
                        out_hbm.at[cid].at[pl.ds(base, SEG)])

    return k(v, edge3)


def _sc_prop(v, edge3, d):
    NBUF = 2 if d > 16 else 5
    MAIN = (CHUNKS // NBUF) * NBUF

    @functools.partial(
        pl.kernel,
        out_type=jax.ShapeDtypeStruct((2, NPAD, d), _f32),
        mesh=_mesh(),
        compiler_params=pltpu.CompilerParams(use_tc_tiling_on_sc=False),
        scratch_types=[
            pltpu.VMEM((CHUNKS, CH), jnp.int32),
            pltpu.VMEM((CHUNKS, CH), jnp.int32),
            [pltpu.VMEM((CH, d), _f32) for _ in range(NBUF)],
            pltpu.VMEM_SHARED((NPAD, d), _f32),
            pltpu.SemaphoreType.DMA((NBUF,)),
            pltpu.SemaphoreType.DMA((NBUF,)),
        ],
    )
    def k(v_hbm, edge_hbm, out_hbm, ridx, cidx, gbufs, acc,
          gsem, ssem):
        cid = lax.axis_index("c")
        sid = lax.axis_index("s")
        wid = cid * NSUB + sid
        pltpu.sync_copy(edge_hbm.at[0].at[wid], ridx)
        pltpu.sync_copy(edge_hbm.at[1].at[wid], cidx)

        def zrow(i, c):
            for t in range(d // 16):
                gbufs[0][i, pl.ds(t * 16, 16)] = jnp.zeros((16,), _f32)
            return c

        lax.fori_loop(0, CH, zrow, 0)
        base = sid * SEG
        for t in range(SEG // CH):
            pltpu.sync_copy(gbufs[0], acc.at[pl.ds(base + t * CH, CH)])
        plsc.subcore_barrier()

        for b in range(NBUF):
            pltpu.async_copy(v_hbm.at[cidx.at[b]], gbufs[b], gsem.at[b])

        def body(i, c):
            j0 = i * NBUF
            for b in range(NBUF):
                pltpu.make_async_copy(v_hbm.at[cidx.at[j0 + b]],
                                      gbufs[b], gsem.at[b]).wait()
                pltpu.async_copy(gbufs[b], acc.at[ridx.at[j0 + b]],
                                 ssem.at[b], add=True)

            @pl.when(j0 + NBUF < MAIN)
            def _():
                for b in range(NBUF):
                    pltpu.make_async_copy(gbufs[b], acc.at[ridx.at[j0 + b]],
                                          ssem.at[b]).wait()
                    pltpu.async_copy(v_hbm.at[cidx.at[j0 + NBUF + b]],
                                     gbufs[b], gsem.at[b])
            return c

        lax.fori_loop(0, MAIN // NBUF, body, 0)
        for b in range(NBUF):
            pltpu.make_async_copy(gbufs[b], acc.at[ridx.at[b]],
                                  ssem.at[b]).wait()
        for j in range(MAIN, CHUNKS):
            pltpu.sync_copy(v_hbm.at[cidx.at[j]], gbufs[0])
            pltpu.sync_copy(gbufs[0], acc.at[ridx.at[j]], add=True)
        plsc.subcore_barrier()
        pltpu.sync_copy(acc.at[pl.ds(base, SEG)],
                        out_hbm.at[cid].at[pl.ds(base, SEG)])

    return k(v, edge3)


def _tc_scale(deg3, x):
    def body(d0, d1, xr, dinv_o, v1_o):
        d = d0[0] + d1[0]
        dinv = lax.rsqrt(d)
        dinv_o[...] = dinv
        v1 = xr[...] * dinv
        v1_o[0] = v1[:, :64]
        v1_o[1] = v1[:, 64:]

    return pl.pallas_call(
        body,
        grid=(N // BLK,),
        in_specs=[
            pl.BlockSpec((1, BLK, 1), lambda j: (0, j, 0)),
            pl.BlockSpec((1, BLK, 1), lambda j: (1, j, 0)),
            pl.BlockSpec((BLK, 128), lambda j: (j, 0)),
        ],
        out_specs=[
            pl.BlockSpec((BLK, 1), lambda j: (j, 0)),
            pl.BlockSpec((2, BLK, 64), lambda j: (0, j, 0)),
        ],
        out_shape=[
            jax.ShapeDtypeStruct((N, 1), _f32),
            jax.ShapeDtypeStruct((2, N, 64), _f32),
        ],
    )(deg3, deg3, x)


def _tc_layer1(u1, dinv, W1, b1, W2):
    def body(p0, p1, s_r, w1_r, b1_r, w2_r, v2_o):
        s = s_r[...]
        p = jnp.concatenate([p0[0], p1[0]], axis=1) * s
        h = jnp.maximum(
            jnp.dot(p, w1_r[...], preferred_element_type=_f32) + b1_r[...], 0.0)
        v2 = jnp.dot(h, w2_r[...], preferred_element_type=_f32) * s
        v2_o[0] = v2[:, :64]
        v2_o[1] = v2[:, 64:]

    return pl.pallas_call(
        body,
        grid=(N // BLK,),
        in_specs=[
            pl.BlockSpec((1, BLK, 64), lambda j: (0, j, 0)),
            pl.BlockSpec((1, BLK, 64), lambda j: (1, j, 0)),
            pl.BlockSpec((BLK, 1), lambda j: (j, 0)),
            pl.BlockSpec((128, 256), lambda j: (0, 0)),
            pl.BlockSpec((1, 256), lambda j: (0, 0)),
            pl.BlockSpec((256, 128), lambda j: (0, 0)),
        ],
        out_specs=pl.BlockSpec((2, BLK, 64), lambda j: (0, j, 0)),
        out_shape=jax.ShapeDtypeStruct((2, N, 64), _f32),
    )(u1, u1, dinv, W1, b1, W2)


def _tc_layer2(u2, dinv, b2, W3):
    def body(p0, p1, s_r, b2_r, w3_r, v3_o):
        s = s_r[...]
        p = jnp.concatenate([p0[0], p1[0]], axis=1)
        h = jnp.maximum(p * s + b2_r[...], 0.0)
        v3_o[...] = jnp.dot(h, w3_r[...], preferred_element_type=_f32) * s

    return pl.pallas_call(
        body,
        grid=(N // BLK,),
        in_specs=[
            pl.BlockSpec((1, BLK, 64), lambda j: (0, j, 0)),
            pl.BlockSpec((1, BLK, 64), lambda j: (1, j, 0)),
            pl.BlockSpec((BLK, 1), lambda j: (j, 0)),
            pl.BlockSpec((1, 128), lambda j: (0, 0)),
            pl.BlockSpec((128, 16), lambda j: (0, 0)),
        ],
        out_specs=pl.BlockSpec((BLK, 16), lambda j: (j, 0)),
        out_shape=jax.ShapeDtypeStruct((N, 16), _f32),
    )(u2, u2, dinv, b2, W3)


def _tc_softmax(u3, dinv, b3):
    def body(p0, p1, s_r, b3_r, o):
        z = (p0[0] + p1[0]) * s_r[...] + b3_r[...]
        z = z - jnp.max(z, axis=1, keepdims=True)
        e = jnp.exp(z)
        o[...] = e / jnp.sum(e, axis=1, keepdims=True)

    return pl.pallas_call(
        body,
        grid=(N // BLK,),
        in_specs=[
            pl.BlockSpec((1, BLK, 16), lambda j: (0, j, 0)),
            pl.BlockSpec((1, BLK, 16), lambda j: (1, j, 0)),
            pl.BlockSpec((BLK, 1), lambda j: (j, 0)),
            pl.BlockSpec((1, 16), lambda j: (0, 0)),
        ],
        out_specs=pl.BlockSpec((BLK, 16), lambda j: (j, 0)),
        out_shape=jax.ShapeDtypeStruct((N, 16), _f32),
    )(u3, u3, dinv, b3)


def kernel(x, edge_index, W1, b1, W2, b2, W3, b3):
    edge3 = edge_index.reshape(2, NW, CHUNKS, CH)
    degs = _sc_degree(edge3)
    deg3 = degs[:, :, None]
    dinv, v1 = _tc_scale(deg3, x)
    u1 = _sc_prop_half(v1, edge3)
    v2 = _tc_layer1(u1, dinv, W1, b1.reshape(1, -1), W2)
    u2 = _sc_prop_half(v2, edge3)
    v3 = _tc_layer2(u2, dinv, b2.reshape(1, -1), W3)
    u3 = _sc_prop(v3, edge3, 16)
    return _tc_softmax(u3, dinv, b3.reshape(1, -1))
